# baseline (speedup 1.0000x reference)
import functools

import jax
import jax.numpy as jnp
from jax import lax
from jax.experimental import pallas as pl
from jax.experimental.pallas import tpu as pltpu
from jax.experimental.pallas import tpu_sc as plsc

HID = 768
EPS = 1e-6
L = 16
NV = HID // L
NC = 2
NS = 16
NW = NC * NS
CHUNK = 32
RGRP = 4


def _lanesum(x):
    lane = lax.iota(jnp.int32, L)
    for m in (1, 2, 4, 8):
        x = x + x.at[lane ^ m].get(mode="promise_in_bounds")
    return x


def _rsqrt16(v):
    i = lax.bitcast_convert_type(v, jnp.int32)
    y = lax.bitcast_convert_type(jnp.int32(0x5F3759DF) - (i >> 1), jnp.float32)
    h = v * 0.5
    for _ in range(3):
        y = y * (1.5 - h * y * y)
    return y


@functools.cache
def _build(n_tokens, seq):
    rows_per_w = n_tokens // NW
    nchunks = rows_per_w // CHUNK
    assert nchunks >= 4 and nchunks % 2 == 0
    mesh = plsc.VectorSubcoreMesh(core_axis_name="c", subcore_axis_name="s")

    @functools.partial(
        pl.kernel,
        mesh=mesh,
        out_type=jax.ShapeDtypeStruct((n_tokens, HID), jnp.float32),
        scratch_types=[
            pltpu.VMEM((rows_per_w,), jnp.int32),
            pltpu.VMEM((CHUNK, HID), jnp.float32),
            pltpu.VMEM((CHUNK, HID), jnp.float32),
            pltpu.VMEM((CHUNK, HID), jnp.float32),
            pltpu.VMEM((CHUNK, HID), jnp.float32),
            pltpu.VMEM((HID,), jnp.float32),
            pltpu.VMEM((HID,), jnp.float32),
            pltpu.SemaphoreType.DMA,
            pltpu.SemaphoreType.DMA,
            pltpu.SemaphoreType.DMA,
            pltpu.SemaphoreType.DMA,
            pltpu.SemaphoreType.DMA,
            pltpu.SemaphoreType.DMA,
        ],
    )
    def k(ids_hbm, word_hbm, pos_hbm, gamma_hbm, beta_hbm, out_hbm,
          idx_v, wbuf0, wbuf1, pbuf0, pbuf1, gv, bv,
          sw0, sw1, sp0, sp1, so0, so1):
        wbuf = (wbuf0, wbuf1)
        pbuf = (pbuf0, pbuf1)
        sw = (sw0, sw1)
        sp = (sp0, sp1)
        so = (so0, so1)

        wid = lax.axis_index("s") * NC + lax.axis_index("c")
        base = wid * rows_per_w
        s0 = base % seq

        pltpu.sync_copy(ids_hbm.at[pl.ds(base, rows_per_w)], idx_v)
        pltpu.sync_copy(gamma_hbm, gv)
        pltpu.sync_copy(beta_hbm, bv)

        def in_copies(c, par):
            row0 = pl.multiple_of(c * CHUNK, CHUNK)
            return (
                pltpu.make_async_copy(pos_hbm.at[pl.ds(s0 + row0, CHUNK)],
                                      pbuf[par], sp[par]),
                pltpu.make_async_copy(word_hbm.at[idx_v.at[pl.ds(row0, CHUNK)]],
                                      wbuf[par], sw[par]),
            )

        def out_copy(c, par):
            row0 = pl.multiple_of(c * CHUNK, CHUNK)
            return pltpu.make_async_copy(
                wbuf[par], out_hbm.at[pl.ds(base + row0, CHUNK)], so[par])

        def compute(par):
            wb = wbuf[par]
            pb = pbuf[par]

            def grp_body(g, gc):
                r0 = pl.multiple_of(g * RGRP, RGRP)
                means, istds = [], []
                for i in range(RGRP):
                    r = r0 + i
                    vsum = jnp.zeros((L,), jnp.float32)
                    vsq = jnp.zeros((L,), jnp.float32)
                    for j in range(NV):
                        sl = pl.ds(j * L, L)
                        x = wb[r, sl] + pb[r, sl]
                        wb[r, sl] = x
                        vsum = vsum + x
                        vsq = vsq + x * x
                    mean_v = _lanesum(vsum) * (1.0 / HID)
                    var_v = _lanesum(vsq) * (1.0 / HID) - mean_v * mean_v
                    means.append(mean_v)
                    istds.append(_rsqrt16(var_v + EPS))
                for j in range(NV):
                    sl = pl.ds(j * L, L)
                    gj = gv[sl]
                    bj = bv[sl]
                    for i in range(RGRP):
                        r = r0 + i
                        x = wb[r, sl]
                        wb[r, sl] = (x - means[i]) * istds[i] * gj + bj
                return gc

            lax.fori_loop(0, CHUNK // RGRP, grp_body, 0)

        def pair_body(c2, carry):
            c = c2 * 2
            @pl.when(c + 1 < nchunks)
            def _pf1():
                @pl.when(c2 > 0)
                def _drain1():
                    out_copy(c - 1, 1).wait()
                for cp in in_copies(c + 1, 1):
                    cp.start()

            for cp in in_copies(c, 0):
                cp.wait()
            compute(0)
            out_copy(c, 0).start()

            @pl.when(c + 2 < nchunks)
            def _pf0():
                out_copy(c, 0).wait()
                for cp in in_copies(c + 2, 0):
                    cp.start()

            for cp in in_copies(c + 1, 1):
                cp.wait()
            compute(1)
            out_copy(c + 1, 1).start()
            return carry

        in_copies(0, 0)[0].start()
        in_copies(0, 0)[1].start()
        lax.fori_loop(0, nchunks // 2, pair_body, 0)
        out_copy(nchunks - 2, 0).wait()
        out_copy(nchunks - 1, 1).wait()

    return k


def kernel(input_ids, word_embeddings, position_embeddings, gamma, beta):
    b, s = input_ids.shape
    ids = input_ids.reshape(-1).astype(jnp.int32)
    out = _build(b * s, s)(ids, word_embeddings, position_embeddings, gamma, beta)
    return out.reshape(b, s, HID)


# speedup vs baseline: 3.6598x; 3.6598x over previous
import functools

import jax
import jax.numpy as jnp
from jax import lax
from jax.experimental import pallas as pl
from jax.experimental.pallas import tpu as pltpu
from jax.experimental.pallas import tpu_sc as plsc

HID = 768
EPS = 1e-6
NC = 2
NS = 16
NW = NC * NS
GCHUNK = 64
TBLK = 256


@functools.cache
def _build_gather(n_tokens):
    rows_per_w = n_tokens // NW
    nchunks = rows_per_w // GCHUNK
    assert nchunks % 2 == 0
    mesh = plsc.VectorSubcoreMesh(core_axis_name="c", subcore_axis_name="s")

    @functools.partial(
        pl.kernel,
        mesh=mesh,
        out_type=jax.ShapeDtypeStruct((n_tokens, HID), jnp.float32),
        scratch_types=[
            pltpu.VMEM((rows_per_w,), jnp.int32),
            pltpu.VMEM((GCHUNK, HID), jnp.float32),
            pltpu.VMEM((GCHUNK, HID), jnp.float32),
            pltpu.SemaphoreType.DMA,
            pltpu.SemaphoreType.DMA,
            pltpu.SemaphoreType.DMA,
            pltpu.SemaphoreType.DMA,
        ],
    )
    def g(ids_hbm, word_hbm, out_hbm, idx_v, b0, b1, sg0, sg1, so0, so1):
        buf = (b0, b1)
        sg = (sg0, sg1)
        so = (so0, so1)
        wid = lax.axis_index("s") * NC + lax.axis_index("c")
        base = wid * rows_per_w

        pltpu.sync_copy(ids_hbm.at[pl.ds(base, rows_per_w)], idx_v)

        def gather(c, par):
            row0 = pl.multiple_of(c * GCHUNK, GCHUNK)
            return pltpu.make_async_copy(
                word_hbm.at[idx_v.at[pl.ds(row0, GCHUNK)]], buf[par], sg[par])

        def drain(c, par):
            row0 = pl.multiple_of(c * GCHUNK, GCHUNK)
            return pltpu.make_async_copy(
                buf[par], out_hbm.at[pl.ds(base + row0, GCHUNK)], so[par])

        gather(0, 0).start()
        gather(1, 1).start()

        def pair_body(c2, carry):
            c = c2 * 2
            gather(c, 0).wait()
            drain(c, 0).start()

            @pl.when(c + 2 < nchunks)
            def _refill0():
                drain(c, 0).wait()
                gather(c + 2, 0).start()

            gather(c + 1, 1).wait()
            drain(c + 1, 1).start()

            @pl.when(c + 3 < nchunks)
            def _refill1():
                drain(c + 1, 1).wait()
                gather(c + 3, 1).start()

            return carry

        lax.fori_loop(0, nchunks // 2, pair_body, 0)
        drain(nchunks - 2, 0).wait()
        drain(nchunks - 1, 1).wait()

    return g


def _ln_body(x_ref, pos_ref, g_ref, b_ref, o_ref):
    x = x_ref[...] + pos_ref[...]
    mean = jnp.mean(x, axis=-1, keepdims=True)
    cen = x - mean
    var = jnp.mean(cen * cen, axis=-1, keepdims=True)
    o_ref[...] = cen * lax.rsqrt(var + EPS) * g_ref[...] + b_ref[...]


@functools.cache
def _build_ln(n_tokens, seq):
    pos_blocks = seq // TBLK
    return pl.pallas_call(
        _ln_body,
        grid=(n_tokens // TBLK,),
        in_specs=[
            pl.BlockSpec((TBLK, HID), lambda i: (i, 0)),
            pl.BlockSpec((TBLK, HID), lambda i: (lax.rem(i, pos_blocks), 0)),
            pl.BlockSpec((HID,), lambda i: (0,)),
            pl.BlockSpec((HID,), lambda i: (0,)),
        ],
        out_specs=pl.BlockSpec((TBLK, HID), lambda i: (i, 0)),
        out_shape=jax.ShapeDtypeStruct((n_tokens, HID), jnp.float32),
    )


def kernel(input_ids, word_embeddings, position_embeddings, gamma, beta):
    b, s = input_ids.shape
    ids = input_ids.reshape(-1).astype(jnp.int32)
    gathered = _build_gather(b * s)(ids, word_embeddings)
    out = _build_ln(b * s, s)(gathered, position_embeddings, gamma, beta)
    return out.reshape(b, s, HID)


# speedup vs baseline: 4.6225x; 1.2631x over previous
import functools

import jax
import jax.numpy as jnp
from jax import lax
from jax.experimental import pallas as pl
from jax.experimental.pallas import tpu as pltpu
from jax.experimental.pallas import tpu_sc as plsc

HID = 768
EPS = 1e-6
NC = 2
NS = 16
NW = NC * NS
GCHUNK = 64
TBLK = 256


@functools.cache
def _build_gather(n_tokens):
    rows_per_w = n_tokens // NW
    nchunks = rows_per_w // GCHUNK
    assert nchunks % 2 == 0
    mesh = plsc.VectorSubcoreMesh(core_axis_name="c", subcore_axis_name="s")

    @functools.partial(
        pl.kernel,
        mesh=mesh,
        out_type=jax.ShapeDtypeStruct((n_tokens, HID), jnp.float32),
        scratch_types=[
            pltpu.VMEM((rows_per_w,), jnp.int32),
            pltpu.VMEM((GCHUNK, HID), jnp.float32),
            pltpu.VMEM((GCHUNK, HID), jnp.float32),
            pltpu.SemaphoreType.DMA,
            pltpu.SemaphoreType.DMA,
            pltpu.SemaphoreType.DMA,
            pltpu.SemaphoreType.DMA,
        ],
    )
    def g(ids_hbm, word_hbm, out_hbm, idx_v, b0, b1, sg0, sg1, so0, so1):
        buf = (b0, b1)
        sg = (sg0, sg1)
        so = (so0, so1)
        wid = lax.axis_index("s") * NC + lax.axis_index("c")
        base = wid * rows_per_w

        pltpu.sync_copy(ids_hbm.at[pl.ds(base, rows_per_w)], idx_v)

        def gather(c, par):
            row0 = pl.multiple_of(c * GCHUNK, GCHUNK)
            return pltpu.make_async_copy(
                word_hbm.at[idx_v.at[pl.ds(row0, GCHUNK)]], buf[par], sg[par])

        def drain(c, par):
            row0 = pl.multiple_of(c * GCHUNK, GCHUNK)
            return pltpu.make_async_copy(
                buf[par], out_hbm.at[pl.ds(base + row0, GCHUNK)], so[par])

        gather(0, 0).start()
        gather(1, 1).start()

        def pair_body(c2, carry):
            c = c2 * 2
            gather(c, 0).wait()
            drain(c, 0).start()

            @pl.when(c + 2 < nchunks)
            def _refill0():
                drain(c, 0).wait()
                gather(c + 2, 0).start()

            gather(c + 1, 1).wait()
            drain(c + 1, 1).start()

            @pl.when(c + 3 < nchunks)
            def _refill1():
                drain(c + 1, 1).wait()
                gather(c + 3, 1).start()

            return carry

        lax.fori_loop(0, nchunks // 2, pair_body, 0)
        drain(nchunks - 2, 0).wait()
        drain(nchunks - 1, 1).wait()

    return g


def _ln_body(x_ref, pos_ref, g_ref, b_ref, o_ref):
    x = x_ref[...] + pos_ref[...][None]
    mean = jnp.mean(x, axis=-1, keepdims=True)
    msq = jnp.mean(x * x, axis=-1, keepdims=True)
    var = msq - mean * mean
    o_ref[...] = (x - mean) * lax.rsqrt(var + EPS) * g_ref[...] + b_ref[...]


@functools.cache
def _build_ln(batch, seq):
    return pl.pallas_call(
        _ln_body,
        grid=(seq // TBLK,),
        in_specs=[
            pl.BlockSpec((batch, TBLK, HID), lambda j: (0, j, 0)),
            pl.BlockSpec((TBLK, HID), lambda j: (j, 0)),
            pl.BlockSpec((HID,), lambda j: (0,)),
            pl.BlockSpec((HID,), lambda j: (0,)),
        ],
        out_specs=pl.BlockSpec((batch, TBLK, HID), lambda j: (0, j, 0)),
        out_shape=jax.ShapeDtypeStruct((batch, seq, HID), jnp.float32),
    )


def kernel(input_ids, word_embeddings, position_embeddings, gamma, beta):
    b, s = input_ids.shape
    ids = input_ids.reshape(-1).astype(jnp.int32)
    gathered = _build_gather(b * s)(ids, word_embeddings)
    return _build_ln(b, s)(gathered.reshape(b, s, HID), position_embeddings,
                           gamma, beta)
